# baseline (speedup 1.0000x reference)
import jax
import jax.numpy as jnp
from jax.experimental import pallas as pl
from jax.experimental.pallas import tpu as pltpu

B = 16384
D = 64
H = 32
G = 18
GP = 20
NG = 5
KG = GP // NG
BT = 8192
W1ROWS = GP * 2 * H
WBROWS = W1ROWS + D

def _fused_kernel(xt_ref, gvt_ref, w1_ref, b1_ref, w2t_ref, b2_ref, w3_ref,
                  b3_ref, wat_ref, bat_ref, wgt_ref, bgt_ref, out_ref,
                  wbigt_s, b1pt_s, w2st_s, b2pt_s, w3rt_s, w3wt_s, b3wt_s):

    @pl.when(pl.program_id(0) == 0)
    def _prep():
        wbigt_s[:] = jnp.zeros((WBROWS, D), jnp.float32)
        b1pt_s[:] = jnp.zeros((W1ROWS, 1), jnp.float32)
        w2st_s[:] = jnp.zeros((NG, KG * H, KG * 2 * H), jnp.float32)
        b2pt_s[:] = jnp.zeros((GP * H, 1), jnp.float32)
        w3rt_s[:] = jnp.zeros((H, GP * H), jnp.float32)
        b1t = jnp.transpose(b1_ref[:])
        b2t = jnp.transpose(b2_ref[:])
        for g in range(G):
            t, i = g // KG, g % KG
            wbigt_s[g * 2 * H:(g + 1) * 2 * H, :] = jnp.transpose(w1_ref[g])
            b1pt_s[g * 2 * H:(g + 1) * 2 * H, :] = b1t[:, g:g + 1]
            w2st_s[t, i * H:(i + 1) * H, i * 2 * H:(i + 1) * 2 * H] = w2t_ref[g]
            b2pt_s[g * H:(g + 1) * H, :] = b2t[:, g:g + 1]
            w3rt_s[:, g * H:(g + 1) * H] = jnp.transpose(w3_ref[g])
        wbigt_s[W1ROWS:, :] = wgt_ref[:, :D]
        wgrt = wgt_ref[:, D:]
        w3wt_s[:] = jnp.dot(wgrt, w3rt_s[:], preferred_element_type=jnp.float32)
        b3wt_s[:] = jnp.dot(wgrt, jnp.transpose(b3_ref[:]),
                            preferred_element_type=jnp.float32)

    xt = xt_ref[:]
    gvt = gvt_ref[:]

    logits = jnp.dot(wat_ref[:], xt, preferred_element_type=jnp.float32) + bat_ref[:]
    m = jnp.max(logits, axis=0, keepdims=True)
    ex = jnp.exp(logits - m)
    gwt = ex / jnp.sum(ex, axis=0, keepdims=True)
    ct = gwt * gvt

    pt = jnp.dot(wbigt_s[:], xt, preferred_element_type=jnp.float32)

    acc = (pt[W1ROWS:, :]
           + jnp.dot(b3wt_s[:], ct, preferred_element_type=jnp.float32)
           + bgt_ref[:])
    for t in range(NG):
        s1 = t * KG * 2 * H
        s2 = t * KG * H
        h1 = jnp.maximum(pt[s1:s1 + KG * 2 * H, :] + b1pt_s[s1:s1 + KG * 2 * H, :], 0.0)
        h2 = jnp.maximum(
            jnp.dot(w2st_s[t], h1, preferred_element_type=jnp.float32)
            + b2pt_s[s2:s2 + KG * H, :], 0.0)
        u = jnp.concatenate(
            [h2[i * H:(i + 1) * H, :] * ct[t * KG + i:t * KG + i + 1, :]
             for i in range(KG) if t * KG + i < G], axis=0)
        acc += jnp.dot(w3wt_s[:, s2:s2 + (KG * H if t < NG - 1 else (G - t * KG) * H)],
                       u, preferred_element_type=jnp.float32)
    out_ref[:] = jnp.maximum(acc, 0.0)


def kernel(item_features, genre_vectors, W1, b1, W2, b2, W3, b3, Wa, ba, Wg, bg):
    xt = jnp.transpose(item_features)
    gvt = jnp.transpose(genre_vectors)
    wat = jnp.transpose(Wa)
    wgt = jnp.transpose(Wg)
    w2t = jnp.transpose(W2, (0, 2, 1))
    bat = ba.reshape(G, 1)
    bgt = bg.reshape(D, 1)

    grid = (B // BT,)
    full = lambda i: (0, 0)
    full3 = lambda i: (0, 0, 0)
    outt = pl.pallas_call(
        _fused_kernel,
        grid=grid,
        in_specs=[
            pl.BlockSpec((D, BT), lambda i: (0, i)),
            pl.BlockSpec((G, BT), lambda i: (0, i)),
            pl.BlockSpec((G, D, 2 * H), full3),
            pl.BlockSpec((G, 2 * H), full),
            pl.BlockSpec((G, H, 2 * H), full3),
            pl.BlockSpec((G, H), full),
            pl.BlockSpec((G, H, H), full3),
            pl.BlockSpec((G, H), full),
            pl.BlockSpec((G, D), full),
            pl.BlockSpec((G, 1), full),
            pl.BlockSpec((D, D + H), full),
            pl.BlockSpec((D, 1), full),
        ],
        out_specs=pl.BlockSpec((D, BT), lambda i: (0, i)),
        out_shape=jax.ShapeDtypeStruct((D, B), jnp.float32),
        scratch_shapes=[
            pltpu.VMEM((WBROWS, D), jnp.float32),
            pltpu.VMEM((W1ROWS, 1), jnp.float32),
            pltpu.VMEM((NG, KG * H, KG * 2 * H), jnp.float32),
            pltpu.VMEM((GP * H, 1), jnp.float32),
            pltpu.VMEM((H, GP * H), jnp.float32),
            pltpu.VMEM((D, GP * H), jnp.float32),
            pltpu.VMEM((D, G), jnp.float32),
        ],
        compiler_params=pltpu.CompilerParams(
            dimension_semantics=("arbitrary",),
        ),
    )(xt, gvt, W1, b1, w2t, b2, W3, b3, wat, bat, wgt, bgt)
    return jnp.transpose(outt)


# speedup vs baseline: 1.0410x; 1.0410x over previous
import jax
import jax.numpy as jnp
from jax.experimental import pallas as pl
from jax.experimental.pallas import tpu as pltpu

B = 16384
D = 64
H = 32
G = 18
GP = 20
NG = 5
KG = GP // NG
BT = 4096
W1ROWS = GP * 2 * H
WBROWS = W1ROWS + D

def _fused_kernel(xt_ref, gvt_ref, w1_ref, b1_ref, w2t_ref, b2_ref, w3_ref,
                  b3_ref, wat_ref, bat_ref, wgt_ref, bgt_ref, out_ref,
                  wbigt_s, b1pt_s, w2st_s, b2pt_s, w3rt_s, w3wt_s, b3wt_s):

    @pl.when(pl.program_id(0) == 0)
    def _prep():
        wbigt_s[:] = jnp.zeros((WBROWS, D), jnp.float32)
        b1pt_s[:] = jnp.zeros((W1ROWS, 1), jnp.float32)
        w2st_s[:] = jnp.zeros((NG, KG * H, KG * 2 * H), jnp.float32)
        b2pt_s[:] = jnp.zeros((GP * H, 1), jnp.float32)
        w3rt_s[:] = jnp.zeros((H, GP * H), jnp.float32)
        b1t = jnp.transpose(b1_ref[:])
        b2t = jnp.transpose(b2_ref[:])
        for g in range(G):
            t, i = g // KG, g % KG
            wbigt_s[g * 2 * H:(g + 1) * 2 * H, :] = jnp.transpose(w1_ref[g])
            b1pt_s[g * 2 * H:(g + 1) * 2 * H, :] = b1t[:, g:g + 1]
            w2st_s[t, i * H:(i + 1) * H, i * 2 * H:(i + 1) * 2 * H] = w2t_ref[g]
            b2pt_s[g * H:(g + 1) * H, :] = b2t[:, g:g + 1]
            w3rt_s[:, g * H:(g + 1) * H] = jnp.transpose(w3_ref[g])
        wbigt_s[W1ROWS:, :] = wgt_ref[:, :D]
        wgrt = wgt_ref[:, D:]
        w3wt_s[:] = jnp.dot(wgrt, w3rt_s[:], preferred_element_type=jnp.float32)
        b3wt_s[:] = jnp.dot(wgrt, jnp.transpose(b3_ref[:]),
                            preferred_element_type=jnp.float32)

    xt = xt_ref[:]
    gvt = gvt_ref[:]

    logits = jnp.dot(wat_ref[:], xt, preferred_element_type=jnp.float32) + bat_ref[:]
    m = jnp.max(logits, axis=0, keepdims=True)
    ex = jnp.exp(logits - m)
    gwt = ex / jnp.sum(ex, axis=0, keepdims=True)
    ct = gwt * gvt

    pt = jnp.dot(wbigt_s[:], xt, preferred_element_type=jnp.float32)

    acc = (pt[W1ROWS:, :]
           + jnp.dot(b3wt_s[:], ct, preferred_element_type=jnp.float32)
           + bgt_ref[:])
    for t in range(NG):
        s1 = t * KG * 2 * H
        s2 = t * KG * H
        h1 = jnp.maximum(pt[s1:s1 + KG * 2 * H, :] + b1pt_s[s1:s1 + KG * 2 * H, :], 0.0)
        h2 = jnp.maximum(
            jnp.dot(w2st_s[t], h1, preferred_element_type=jnp.float32)
            + b2pt_s[s2:s2 + KG * H, :], 0.0)
        u = jnp.concatenate(
            [h2[i * H:(i + 1) * H, :] * ct[t * KG + i:t * KG + i + 1, :]
             for i in range(KG) if t * KG + i < G], axis=0)
        acc += jnp.dot(w3wt_s[:, s2:s2 + (KG * H if t < NG - 1 else (G - t * KG) * H)],
                       u, preferred_element_type=jnp.float32)
    out_ref[:] = jnp.maximum(acc, 0.0)


def kernel(item_features, genre_vectors, W1, b1, W2, b2, W3, b3, Wa, ba, Wg, bg):
    xt = jnp.transpose(item_features)
    gvt = jnp.transpose(genre_vectors)
    wat = jnp.transpose(Wa)
    wgt = jnp.transpose(Wg)
    w2t = jnp.transpose(W2, (0, 2, 1))
    bat = ba.reshape(G, 1)
    bgt = bg.reshape(D, 1)

    grid = (B // BT,)
    full = lambda i: (0, 0)
    full3 = lambda i: (0, 0, 0)
    outt = pl.pallas_call(
        _fused_kernel,
        grid=grid,
        in_specs=[
            pl.BlockSpec((D, BT), lambda i: (0, i)),
            pl.BlockSpec((G, BT), lambda i: (0, i)),
            pl.BlockSpec((G, D, 2 * H), full3),
            pl.BlockSpec((G, 2 * H), full),
            pl.BlockSpec((G, H, 2 * H), full3),
            pl.BlockSpec((G, H), full),
            pl.BlockSpec((G, H, H), full3),
            pl.BlockSpec((G, H), full),
            pl.BlockSpec((G, D), full),
            pl.BlockSpec((G, 1), full),
            pl.BlockSpec((D, D + H), full),
            pl.BlockSpec((D, 1), full),
        ],
        out_specs=pl.BlockSpec((D, BT), lambda i: (0, i)),
        out_shape=jax.ShapeDtypeStruct((D, B), jnp.float32),
        scratch_shapes=[
            pltpu.VMEM((WBROWS, D), jnp.float32),
            pltpu.VMEM((W1ROWS, 1), jnp.float32),
            pltpu.VMEM((NG, KG * H, KG * 2 * H), jnp.float32),
            pltpu.VMEM((GP * H, 1), jnp.float32),
            pltpu.VMEM((H, GP * H), jnp.float32),
            pltpu.VMEM((D, GP * H), jnp.float32),
            pltpu.VMEM((D, G), jnp.float32),
        ],
        compiler_params=pltpu.CompilerParams(
            dimension_semantics=("arbitrary",),
        ),
    )(xt, gvt, W1, b1, w2t, b2, W3, b3, wat, bat, wgt, bgt)
    return jnp.transpose(outt)
